# baseline (speedup 1.0000x reference)
import functools

import jax
import jax.numpy as jnp
from jax import lax
from jax.experimental import pallas as pl
from jax.experimental.pallas import tpu as pltpu
from jax.experimental.pallas import tpu_sc as plsc

_BATCH = 4096
_SEQ = 200
_D = 64
_DW = _D // 2
_H = 128
_NW = 32
_BPW = _BATCH // _NW
_CHUNK = 40
_NCHUNK = _SEQ // _CHUNK
_NCH_W = _BPW * _NCHUNK
_JW = _BPW * _SEQ
_PW = _NCH_W * 64


def _pool_body(xj_hbm, xp_hbm, table_hbm, out_hbm,
               j_v, p_v, rows_v, out_v, sem0, sem1):
    wid = lax.axis_index("s") * 2 + lax.axis_index("c")
    pltpu.sync_copy(xj_hbm.at[pl.ds(wid * _JW, _JW)], j_v)
    pltpu.sync_copy(xp_hbm.at[pl.ds(wid * _PW, _PW)], p_v)

    sems = (sem0, sem1)

    def issue(chunk, buf):
        pltpu.async_copy(
            table_hbm.at[j_v.at[pl.ds(_CHUNK * chunk, _CHUNK)]],
            rows_v.at[buf], sems[buf])

    def wait(chunk, buf):
        pltpu.make_async_copy(
            table_hbm.at[j_v.at[pl.ds(_CHUNK * chunk, _CHUNK)]],
            rows_v.at[buf], sems[buf]).wait()

    issue(0, 0)
    issue(1, 1)

    neg_inf = jnp.full((32,), -jnp.inf, jnp.bfloat16)

    def reduce_chunk(chunk, buf, acc):
        pbase = pl.multiple_of(64 * chunk, 16)

        def group(acc, seg, jbase):
            pvec = p_v[pl.ds(pbase + 16 * seg, 16)]
            a0, a1 = acc
            for t in range(16):
                ps = pl.multiple_of(pvec[t], 16)
                j = jbase + t
                lo = rows_v[buf, j, pl.ds(ps, 16)]
                hi = rows_v[buf, j, pl.ds(ps + 16, 16)]
                a0 = jnp.maximum(a0, plsc.bitcast(lo, jnp.bfloat16))
                a1 = jnp.maximum(a1, plsc.bitcast(hi, jnp.bfloat16))
            return (a0, a1)

        acc = group(acc, 0, 0)
        acc = group(acc, 1, 16)
        return group(acc, 2, 24)

    def body(g, carry):
        acc = (neg_inf, neg_inf)
        for k in range(2 * _NCHUNK):
            buf = k % 2
            chunk = 2 * _NCHUNK * g + k
            if k % _NCHUNK == 0:
                acc = (neg_inf, neg_inf)
            wait(chunk, buf)
            acc = reduce_chunk(chunk, buf, acc)

            @pl.when(chunk + 2 < _NCH_W)
            def _():
                issue(chunk + 2, buf)

            if k % _NCHUNK == _NCHUNK - 1:
                row = 2 * g + k // _NCHUNK
                base = pl.multiple_of(row * _DW, 16)
                out_v[pl.ds(base, 16)] = plsc.bitcast(acc[0], jnp.int32)
                out_v[pl.ds(base + 16, 16)] = plsc.bitcast(acc[1], jnp.int32)
        return carry

    lax.fori_loop(0, _BPW // 2, body, 0)

    pltpu.sync_copy(out_v, out_hbm.at[pl.ds(wid * _BPW * _DW, _BPW * _DW)])


_pool = functools.partial(
    pl.kernel,
    out_type=jax.ShapeDtypeStruct((_BATCH * _DW,), jnp.int32),
    mesh=plsc.VectorSubcoreMesh(core_axis_name="c", subcore_axis_name="s"),
    scratch_types=[
        pltpu.VMEM((_JW,), jnp.int32),
        pltpu.VMEM((_PW,), jnp.int32),
        pltpu.VMEM((2, _CHUNK, 4 * _DW), jnp.int32),
        pltpu.VMEM((_BPW * _DW,), jnp.int32),
        pltpu.SemaphoreType.DMA,
        pltpu.SemaphoreType.DMA,
    ],
    compiler_params=pltpu.CompilerParams(needs_layout_passes=False),
)(_pool_body)


def _linear_norm_body(p_ref, wt_ref, b_ref, o_ref):
    h = jnp.dot(p_ref[...].astype(jnp.float32), wt_ref[...],
                preferred_element_type=jnp.float32) + b_ref[...]
    nrm = jnp.sqrt(jnp.sum(h * h, axis=1, keepdims=True))
    o_ref[...] = h / jnp.maximum(nrm, 1e-12)


def kernel(x, embed_table, W, b):
    x32 = x.astype(jnp.int32)
    xj = (x32 >> 2).reshape(-1)
    xp40 = ((x32 & 3) << 5).reshape(-1, _CHUNK)
    xp = jnp.concatenate(
        [xp40[:, 0:32], xp40[:, 24:40],
         jnp.zeros((xp40.shape[0], 16), jnp.int32)], axis=1).reshape(-1)
    tb = embed_table.astype(jnp.bfloat16)
    table3 = lax.bitcast_convert_type(
        tb.reshape(tb.shape[0] // 4, 4 * _DW, 2), jnp.int32)
    pooled_i32 = _pool(xj, xp, table3)
    pooled = lax.bitcast_convert_type(
        pooled_i32.reshape(_BATCH, _DW), jnp.bfloat16).reshape(_BATCH, _D)

    grid = 8
    blk = _BATCH // grid
    out = pl.pallas_call(
        _linear_norm_body,
        out_shape=jax.ShapeDtypeStruct((_BATCH, _H), jnp.float32),
        grid=(grid,),
        in_specs=[
            pl.BlockSpec((blk, _D), lambda i: (i, 0)),
            pl.BlockSpec((_D, _H), lambda i: (0, 0)),
            pl.BlockSpec((1, _H), lambda i: (0, 0)),
        ],
        out_specs=pl.BlockSpec((blk, _H), lambda i: (i, 0)),
    )(pooled, W.T, b[None, :])
    return out
